# baseline (speedup 1.0000x reference)
import jax
import jax.numpy as jnp
from jax import lax
from jax.experimental import pallas as pl
from jax.experimental.pallas import tpu as pltpu
from jax.experimental.pallas import tpu_sc as plsc

NC = 2
NS = 16
NW = NC * NS

SEQ = 200
BATCH = 4096
DIM = 64
NBT = BATCH // 128
NTASK = SEQ * NBT
TPW = NTASK // NW
IPW = TPW * 128


def _emb_kernel(x_hbm, table_hbm, out_hbm,
                idx_v, hv_v, rows0, rows1, ob0, ob1,
                sem_g0, sem_g1, sem_s0, sem_s1):
    wid = lax.axis_index("s") * NC + lax.axis_index("c")
    tbase = wid * TPW

    pltpu.sync_copy(x_hbm.at[pl.ds(wid * IPW, IPW)], idx_v)

    def prep(j, carry):
        v = idx_v[pl.ds(j * 16, 16)]
        hv_v[pl.ds(j * 16, 16)] = (v & 1) * 64
        idx_v[pl.ds(j * 16, 16)] = lax.shift_right_logical(v, 1)
        return carry

    lax.fori_loop(0, IPW // 16, prep, 0)

    iota16 = lax.broadcasted_iota(jnp.int32, (16,), 0)

    def gather(t, rows, sem):
        return pltpu.async_copy(
            table_hbm.at[idx_v.at[pl.ds(t * 128, 128)]], rows, sem)

    def wait_gather(rows, sem):
        pltpu.make_async_copy(
            table_hbm.at[idx_v.at[pl.ds(0, 128)]], rows, sem).wait()

    def transpose(t, rows, ob):
        for bg in range(8):
            hvec = hv_v[pl.ds(t * 128 + bg * 16, 16)]
            rowvec = iota16 + (bg * 16)
            for d in range(DIM):
                val = plsc.load_gather(rows, [rowvec, hvec + d])
                ob[d // 8, d % 8, pl.ds(bg * 16, 16)] = val

    def store(t, ob, sem):
        tt = tbase + t
        s = tt // NBT
        bt = tt % NBT
        return pltpu.async_copy(ob, out_hbm.at[s, :, bt], sem)

    def wait_store(ob, sem):
        pltpu.make_async_copy(ob, out_hbm.at[0, :, 0], sem).wait()

    gather(0, rows0, sem_g0)

    def body(i, carry):
        t0 = 2 * i
        t1 = t0 + 1
        gather(t1, rows1, sem_g1)
        wait_gather(rows0, sem_g0)

        @pl.when(i > 0)
        def _():
            wait_store(ob0, sem_s0)

        transpose(t0, rows0, ob0)
        store(t0, ob0, sem_s0)

        @pl.when(i < TPW // 2 - 1)
        def _():
            gather(t0 + 2, rows0, sem_g0)

        wait_gather(rows1, sem_g1)

        @pl.when(i > 0)
        def _():
            wait_store(ob1, sem_s1)

        transpose(t1, rows1, ob1)
        store(t1, ob1, sem_s1)
        return carry

    lax.fori_loop(0, TPW // 2, body, 0)
    wait_store(ob0, sem_s0)
    wait_store(ob1, sem_s1)


def kernel(x, table):
    xT = jnp.transpose(x).reshape(-1)
    tableP = table.reshape(500000, 128)
    out5 = pl.kernel(
        _emb_kernel,
        out_type=jax.ShapeDtypeStruct((SEQ, 8, NBT, 8, 128), jnp.float32),
        mesh=plsc.VectorSubcoreMesh(core_axis_name="c", subcore_axis_name="s"),
        compiler_params=pltpu.CompilerParams(
            use_tc_tiling_on_sc=False, needs_layout_passes=False),
        scratch_types=[
            pltpu.VMEM((IPW,), jnp.int32),
            pltpu.VMEM((IPW,), jnp.int32),
            pltpu.VMEM((128, 128), jnp.float32),
            pltpu.VMEM((128, 128), jnp.float32),
            pltpu.VMEM((8, 8, 128), jnp.float32),
            pltpu.VMEM((8, 8, 128), jnp.float32),
            pltpu.SemaphoreType.DMA,
            pltpu.SemaphoreType.DMA,
            pltpu.SemaphoreType.DMA,
            pltpu.SemaphoreType.DMA,
        ],
    )(xT, tableP)
    return out5.transpose(2, 4, 0, 1, 3).reshape(BATCH, SEQ, DIM)


# speedup vs baseline: 1.5153x; 1.5153x over previous
import jax
import jax.numpy as jnp
from jax import lax
from jax.experimental import pallas as pl
from jax.experimental.pallas import tpu as pltpu
from jax.experimental.pallas import tpu_sc as plsc

NC = 2
NS = 16
NW = NC * NS

SEQ = 200
BATCH = 4096
DIM = 64
NBT = BATCH // 128
NTASK = SEQ * NBT
TPW = NTASK // NW
IPW = TPW * 128


def _emb_kernel(x_hbm, table_hbm, out_hbm,
                idx_v, hv_v, rows0, rows1, ob0, ob1,
                sem_g0, sem_g1, sem_s0, sem_s1):
    wid = lax.axis_index("s") * NC + lax.axis_index("c")
    tbase = wid * TPW

    pltpu.sync_copy(x_hbm.at[pl.ds(wid * IPW, IPW)], idx_v)

    def prep(j, carry):
        v = idx_v[pl.ds(j * 16, 16)]
        hv_v[pl.ds(j * 16, 16)] = (v & 1) * 64
        idx_v[pl.ds(j * 16, 16)] = lax.shift_right_logical(v, 1)
        return carry

    lax.fori_loop(0, IPW // 16, prep, 0)

    iota16 = lax.broadcasted_iota(jnp.int32, (16,), 0)

    def gather(t, rows, sem):
        return pltpu.async_copy(
            table_hbm.at[idx_v.at[pl.ds(t * 128, 128)]], rows, sem)

    def wait_gather(rows, sem):
        pltpu.make_async_copy(
            table_hbm.at[idx_v.at[pl.ds(0, 128)]], rows, sem).wait()

    def transpose(t, rows, ob):
        for bg in range(8):
            hvec = hv_v[pl.ds(t * 128 + bg * 16, 16)]
            rowvec = iota16 + (bg * 16)

            @plsc.parallel_loop(0, DIM, 1, unroll=8)
            def _(d):
                dt = lax.div(d, 8)
                dr = lax.rem(d, 8)
                val = plsc.load_gather(rows, [rowvec, hvec + d])
                ob[dt, dr, pl.ds(bg * 16, 16)] = val

    def store(t, ob, sem):
        tt = tbase + t
        s = tt // NBT
        bt = tt % NBT
        return pltpu.async_copy(ob, out_hbm.at[s, :, bt], sem)

    def wait_store(ob, sem):
        pltpu.make_async_copy(ob, out_hbm.at[0, :, 0], sem).wait()

    gather(0, rows0, sem_g0)

    def body(i, carry):
        t0 = 2 * i
        t1 = t0 + 1
        gather(t1, rows1, sem_g1)
        wait_gather(rows0, sem_g0)

        @pl.when(i > 0)
        def _():
            wait_store(ob0, sem_s0)

        transpose(t0, rows0, ob0)
        store(t0, ob0, sem_s0)

        @pl.when(i < TPW // 2 - 1)
        def _():
            gather(t0 + 2, rows0, sem_g0)

        wait_gather(rows1, sem_g1)

        @pl.when(i > 0)
        def _():
            wait_store(ob1, sem_s1)

        transpose(t1, rows1, ob1)
        store(t1, ob1, sem_s1)
        return carry

    lax.fori_loop(0, TPW // 2, body, 0)
    wait_store(ob0, sem_s0)
    wait_store(ob1, sem_s1)


def kernel(x, table):
    xT = jnp.transpose(x).reshape(-1)
    tableP = table.reshape(500000, 128)
    out5 = pl.kernel(
        _emb_kernel,
        out_type=jax.ShapeDtypeStruct((SEQ, 8, NBT, 8, 128), jnp.float32),
        mesh=plsc.VectorSubcoreMesh(core_axis_name="c", subcore_axis_name="s"),
        compiler_params=pltpu.CompilerParams(
            use_tc_tiling_on_sc=False, needs_layout_passes=False),
        scratch_types=[
            pltpu.VMEM((IPW,), jnp.int32),
            pltpu.VMEM((IPW,), jnp.int32),
            pltpu.VMEM((128, 128), jnp.float32),
            pltpu.VMEM((128, 128), jnp.float32),
            pltpu.VMEM((8, 8, 128), jnp.float32),
            pltpu.VMEM((8, 8, 128), jnp.float32),
            pltpu.SemaphoreType.DMA,
            pltpu.SemaphoreType.DMA,
            pltpu.SemaphoreType.DMA,
            pltpu.SemaphoreType.DMA,
        ],
    )(xT, tableP)
    return out5.transpose(2, 4, 0, 1, 3).reshape(BATCH, SEQ, DIM)


# speedup vs baseline: 1.5176x; 1.0015x over previous
import jax
import jax.numpy as jnp
from jax import lax
from jax.experimental import pallas as pl
from jax.experimental.pallas import tpu as pltpu
from jax.experimental.pallas import tpu_sc as plsc

NC = 2
NS = 16
NW = NC * NS

SEQ = 200
BATCH = 4096
DIM = 64
NBT = BATCH // 128
NTASK = SEQ * NBT
TPW = NTASK // NW
IPW = TPW * 128


def _emb_kernel(x_hbm, table_hbm, out_hbm,
                idx_v, hv_v, rows0, rows1, ob0, ob1,
                sem_g0, sem_g1, sem_s0, sem_s1):
    wid = lax.axis_index("s") * NC + lax.axis_index("c")
    tbase = wid * TPW

    pltpu.sync_copy(x_hbm.at[pl.ds(wid * IPW, IPW)], idx_v)

    def prep(j, carry):
        v = idx_v[pl.ds(j * 16, 16)]
        hv_v[pl.ds(j * 16, 16)] = (v & 1) * 64
        idx_v[pl.ds(j * 16, 16)] = lax.shift_right_logical(v, 1)
        return carry

    lax.fori_loop(0, IPW // 16, prep, 0)

    iota16 = lax.broadcasted_iota(jnp.int32, (16,), 0)

    def gather(t, rows, sem):
        return pltpu.async_copy(
            table_hbm.at[idx_v.at[pl.ds(t * 128, 128)]], rows, sem)

    def wait_gather(rows, sem):
        pltpu.make_async_copy(
            table_hbm.at[idx_v.at[pl.ds(0, 128)]], rows, sem).wait()

    def transpose(t, rows, ob):
        for bg in range(8):
            hvec = hv_v[pl.ds(t * 128 + bg * 16, 16)]
            rowvec = iota16 + (bg * 16)

            @plsc.parallel_loop(0, DIM, 1, unroll=8)
            def _(d):
                dt = lax.div(d, 8)
                dr = lax.rem(d, 8)
                val = plsc.load_gather(rows, [rowvec, hvec + d])
                ob[dt, dr, pl.ds(bg * 16, 16)] = val

    def store(t, ob, sem):
        tt = tbase + t
        s = tt // NBT
        bt = tt % NBT
        return pltpu.async_copy(ob, out_hbm.at[s, :, bt], sem)

    def wait_store(ob, sem):
        pltpu.make_async_copy(ob, out_hbm.at[0, :, 0], sem).wait()

    gather(0, rows0, sem_g0)

    def body(i, carry):
        t0 = 2 * i
        t1 = t0 + 1
        gather(t1, rows1, sem_g1)
        wait_gather(rows0, sem_g0)

        @pl.when(i > 0)
        def _():
            wait_store(ob0, sem_s0)

        transpose(t0, rows0, ob0)
        store(t0, ob0, sem_s0)

        @pl.when(i < TPW // 2 - 1)
        def _():
            gather(t0 + 2, rows0, sem_g0)

        wait_gather(rows1, sem_g1)

        @pl.when(i > 0)
        def _():
            wait_store(ob1, sem_s1)

        transpose(t1, rows1, ob1)
        store(t1, ob1, sem_s1)
        return carry

    lax.fori_loop(0, TPW // 2, body, 0)
    wait_store(ob0, sem_s0)
    wait_store(ob1, sem_s1)


def kernel(x, table):
    xT = jnp.transpose(x).reshape(-1)
    tableP = table.reshape(500000, 128)
    out5 = pl.kernel(
        _emb_kernel,
        out_type=jax.ShapeDtypeStruct((SEQ, 8, NBT, 8, 128), jnp.float32),
        mesh=plsc.VectorSubcoreMesh(core_axis_name="c", subcore_axis_name="s"),
        compiler_params=pltpu.CompilerParams(
            use_tc_tiling_on_sc=True, needs_layout_passes=False),
        scratch_types=[
            pltpu.VMEM((IPW,), jnp.int32),
            pltpu.VMEM((IPW,), jnp.int32),
            pltpu.VMEM((128, 128), jnp.float32),
            pltpu.VMEM((128, 128), jnp.float32),
            pltpu.VMEM((8, 8, 128), jnp.float32),
            pltpu.VMEM((8, 8, 128), jnp.float32),
            pltpu.SemaphoreType.DMA,
            pltpu.SemaphoreType.DMA,
            pltpu.SemaphoreType.DMA,
            pltpu.SemaphoreType.DMA,
        ],
    )(xT, tableP)
    return out5.transpose(2, 4, 0, 1, 3).reshape(BATCH, SEQ, DIM)


# speedup vs baseline: 1.5225x; 1.0032x over previous
import jax
import jax.numpy as jnp
from jax import lax
from jax.experimental import pallas as pl
from jax.experimental.pallas import tpu as pltpu
from jax.experimental.pallas import tpu_sc as plsc

NC = 2
NS = 16
NW = NC * NS

SEQ = 200
BATCH = 4096
DIM = 64
NBT = BATCH // 128
NTASK = SEQ * NBT
TPW = NTASK // NW
IPW = TPW * 128


NB = 3


def _emb_kernel(x_hbm, table_hbm, out_hbm,
                idx_v, idxp_v, rows, obs, sem_g, sem_s):
    wid = lax.axis_index("s") * NC + lax.axis_index("c")
    tbase = wid * TPW

    pltpu.sync_copy(x_hbm.at[pl.ds(wid * IPW, IPW)], idx_v)

    @plsc.parallel_loop(0, IPW // 16, 1, unroll=8)
    def _(j):
        v = idx_v[pl.ds(j * 16, 16)]
        idxp_v[pl.ds(j * 16, 16)] = lax.shift_right_logical(v, 1)

    iota16 = lax.broadcasted_iota(jnp.int32, (16,), 0)

    def gather(t, b):
        return pltpu.async_copy(
            table_hbm.at[idxp_v.at[pl.ds(t * 128, 128)]], rows[b], sem_g[b])

    def wait_gather(b):
        pltpu.make_async_copy(
            table_hbm.at[idxp_v.at[pl.ds(0, 128)]], rows[b], sem_g[b]).wait()

    def transpose(t, b):
        for bg in range(8):
            hvec = (idx_v[pl.ds(t * 128 + bg * 16, 16)] & 1) * 64
            rowvec = iota16 + (bg * 16)

            @plsc.parallel_loop(0, DIM, 1, unroll=8)
            def _(d):
                dt = lax.div(d, 8)
                dr = lax.rem(d, 8)
                val = plsc.load_gather(rows[b], [rowvec, hvec + d])
                obs[b][dt, dr, pl.ds(bg * 16, 16)] = val

    def store(t, b):
        tt = tbase + t
        s = tt // NBT
        bt = tt % NBT
        return pltpu.async_copy(obs[b], out_hbm.at[s, :, bt], sem_s[b])

    def wait_store(b):
        pltpu.make_async_copy(obs[b], out_hbm.at[0, :, 0], sem_s[b]).wait()

    for b in range(NB):
        gather(b, b)

    def body(i, carry):
        t0 = NB * i
        for b in range(NB):
            t = t0 + b
            wait_gather(b)

            @pl.when(i > 0)
            def _():
                wait_store(b)

            transpose(t, b)
            store(t, b)

            @pl.when(t + NB < TPW)
            def _():
                gather(t + NB, b)
        return carry

    lax.fori_loop(0, TPW // NB, body, 0)
    for b in range(TPW % NB):
        t = (TPW // NB) * NB + b
        wait_gather(b)
        wait_store(b)
        transpose(t, b)
        store(t, b)
    for b in range(NB):
        wait_store(b)


def kernel(x, table):
    xT = jnp.transpose(x).reshape(-1)
    tableP = table.reshape(500000, 128)
    out5 = pl.kernel(
        _emb_kernel,
        out_type=jax.ShapeDtypeStruct((SEQ, 8, NBT, 8, 128), jnp.float32),
        mesh=plsc.VectorSubcoreMesh(core_axis_name="c", subcore_axis_name="s"),
        compiler_params=pltpu.CompilerParams(
            use_tc_tiling_on_sc=True, needs_layout_passes=False),
        scratch_types=[
            pltpu.VMEM((IPW,), jnp.int32),
            pltpu.VMEM((IPW,), jnp.int32),
            [pltpu.VMEM((128, 128), jnp.float32) for _ in range(NB)],
            [pltpu.VMEM((8, 8, 128), jnp.float32) for _ in range(NB)],
            [pltpu.SemaphoreType.DMA for _ in range(NB)],
            [pltpu.SemaphoreType.DMA for _ in range(NB)],
        ],
    )(xT, tableP)
    return out5.transpose(2, 4, 0, 1, 3).reshape(BATCH, SEQ, DIM)
